# baseline (speedup 1.0000x reference)
import functools

import jax
import jax.numpy as jnp
from jax import lax
from jax.experimental import pallas as pl
from jax.experimental.pallas import tpu as pltpu
from jax.experimental.pallas import tpu_sc as plsc

NUM_VOCAB = 1000000
EMBED_DIM = 64
ROW = 128
HIDDEN_DIM = 256
BATCH = 4096
SEQ = 200

_TBLK = 8192
_PHASE = 507904
_SPLITP = 253952
_NBLK = 31
_LASTBLK = NUM_VOCAB // _TBLK

_INFO = plsc.get_sparse_core_info()
_NC = _INFO.num_cores
_NS = _INFO.num_subcores
_NW = _NC * _NS
_BPW = BATCH // _NW
_IPW = _BPW * SEQ
_LIST = 14080
_VPE = SEQ // 16 + 1
_ECH = 8
_RAWCH = _ECH * SEQ


def _repack_body(x1_ref, x2_ref, o_ref):
  eye = jnp.asarray(
      lax.broadcasted_iota(jnp.int32, (EMBED_DIM, EMBED_DIM), 0)
      == lax.broadcasted_iota(jnp.int32, (EMBED_DIM, EMBED_DIM), 1),
      jnp.float32,
  )
  dims = (((0,), (0,)), ((), ()))
  o_ref[:, 0:EMBED_DIM] = lax.dot_general(
      x1_ref[...], eye, dims, preferred_element_type=jnp.float32
  )
  o_ref[:, EMBED_DIM:ROW] = lax.dot_general(
      x2_ref[...], eye, dims, preferred_element_type=jnp.float32
  )


def _tc_repack(tableT, blk0):
  return pl.pallas_call(
      _repack_body,
      grid=(_NBLK,),
      in_specs=[
          pl.BlockSpec((EMBED_DIM, _TBLK), lambda i: (0, blk0 + i)),
          pl.BlockSpec(
              (EMBED_DIM, _TBLK),
              lambda i: (0, jnp.minimum(blk0 + _NBLK + i, _LASTBLK)),
          ),
      ],
      out_specs=pl.BlockSpec((_TBLK, ROW), lambda i: (i, 0)),
      out_shape=jax.ShapeDtypeStruct((_SPLITP, ROW), jnp.float32),
      compiler_params=pltpu.CompilerParams(fuse_transposed_lhs_in_matmul=True),
  )(tableT, tableT)


def _sc_pool_phase(xf, tp, second):
  mesh = plsc.VectorSubcoreMesh(core_axis_name="c", subcore_axis_name="s")

  @functools.partial(
      pl.kernel,
      out_type=jax.ShapeDtypeStruct((BATCH, EMBED_DIM), jnp.float32),
      mesh=mesh,
      scratch_types=[
          pltpu.VMEM((_RAWCH + 16,), jnp.int32),
          pltpu.VMEM((_LIST,), jnp.int32),
          pltpu.VMEM((_LIST,), jnp.int32),
          pltpu.VMEM((ROW, ROW), jnp.float32),
          pltpu.VMEM((ROW, ROW), jnp.float32),
          pltpu.VMEM((4 * (_BPW + 1), EMBED_DIM), jnp.float32),
          pltpu.SemaphoreType.DMA,
          pltpu.SemaphoreType.DMA,
      ],
      compiler_params=pltpu.CompilerParams(
          use_tc_tiling_on_sc=True, needs_layout_passes=False
      ),
  )
  def k(xf_hbm, tp_hbm, out_hbm, raw_v, gl_v, eb_v, rows0, rows1, out_v,
        sem0, sem1):
    wid = lax.axis_index("s") * _NC + lax.axis_index("c")
    base = wid * _BPW
    bufs = ((rows0, sem0), (rows1, sem1))
    zero16 = jnp.zeros((16,), jnp.float32)
    zi16 = jnp.zeros((16,), jnp.int32)
    dump16 = jnp.full((16,), _BPW, jnp.int32)

    @plsc.parallel_loop(0, 4 * (_BPW + 1))
    def _(r):
      for g in range(4):
        out_v[r, pl.ds(g * 16, 16)] = zero16

    lanes = lax.iota(jnp.int32, 16)

    def chunk_compress(cb, off):
      pltpu.sync_copy(xf_hbm.at[pl.ds(base * SEQ + cb * _RAWCH, _RAWCH)],
                      raw_v.at[pl.ds(0, _RAWCH)])
      for b2 in range(_ECH):
        b = cb * _ECH + b2
        for j in range(_VPE):
          v = raw_v[pl.ds(b2 * SEQ + j * 16, 16)]
          if second:
            m = v >= _PHASE
            vv = v - _PHASE
          else:
            m = v < _PHASE
            vv = v
          if j == _VPE - 1:
            m = m & (lanes < SEQ - (_VPE - 1) * 16)
          sec = vv >= _SPLITP
          idx2 = jnp.where(sec, vv - _SPLITP, vv)
          w = b + jnp.where(sec, EMBED_DIM << 8, 0)
          plsc.store_compressed(gl_v.at[pl.ds(off, 16)], idx2, mask=m)
          plsc.store_compressed(eb_v.at[pl.ds(off, 16)], w, mask=m)
          off = off + plsc.all_reduce_population_count(m)[0]
        gl_v[pl.ds(off, 16)] = zi16
        eb_v[pl.ds(off, 16)] = dump16
        off = jnp.bitwise_and(off + 7, -8)
      return off

    nidx = lax.fori_loop(0, _BPW // _ECH, chunk_compress, jnp.int32(0))
    for t in range(8):
      gl_v[pl.ds(nidx + t * 16, 16)] = zi16
      eb_v[pl.ds(nidx + t * 16, 16)] = dump16
    nst = (nidx + ROW - 1) // ROW

    def start(k_, rows_v, sem):
      pltpu.async_copy(
          tp_hbm.at[gl_v.at[pl.ds(k_ * ROW, ROW)]], rows_v, sem
      )

    def finish(k_, rows_v, sem):
      pltpu.make_async_copy(
          tp_hbm.at[gl_v.at[pl.ds(k_ * ROW, ROW)]], rows_v, sem
      ).wait()

      @plsc.parallel_loop(0, ROW, unroll=8)
      def _(r):
        w = eb_v[pl.ds(k_ * ROW + r, 16)]
        orow = (w[0] & 255) * 4 + (r & 3)
        h = w[0] >> 8
        for g in range(4):
          plsc.addupdate(
              out_v.at[orow, pl.ds(g * 16, 16)],
              rows_v[r, pl.ds(h + g * 16, 16)],
          )

    for b in range(2):
      @pl.when(b < nst)
      def _():
        start(b, *bufs[b])

    def stream_body(g, carry):
      for b in range(2):
        k_ = 2 * g + b
        rows_v, sem = bufs[b]

        @pl.when(k_ < nst)
        def _():
          finish(k_, rows_v, sem)

        @pl.when(k_ + 2 < nst)
        def _():
          start(k_ + 2, rows_v, sem)

      return carry

    lax.fori_loop(0, (_LIST // ROW + 1) // 2, stream_body, 0)

    def fold_body(r, carry):
      for g in range(4):
        s = pl.ds(g * 16, 16)
        out_v[r, s] = (out_v[4 * r, s] + out_v[4 * r + 1, s]
                       + out_v[4 * r + 2, s] + out_v[4 * r + 3, s])
      return carry

    lax.fori_loop(0, _BPW, fold_body, 0)
    pltpu.sync_copy(out_v.at[pl.ds(0, _BPW)], out_hbm.at[pl.ds(base, _BPW)])

  return k(xf, tp)


def _mlp_body(p0_ref, p1_ref, w1_ref, b1_ref, w2_ref, b2_ref, o_ref):
  h = (p0_ref[...] + p1_ref[...]) * (1.0 / SEQ)
  h1 = lax.dot_general(
      h, w1_ref[...], (((1,), (1,)), ((), ())),
      preferred_element_type=jnp.float32,
  )
  h1 = jnp.maximum(h1 + b1_ref[...], 0.0)
  o = jnp.sum(h1 * w2_ref[...], axis=1, keepdims=True) + b2_ref[...]
  o_ref[...] = 1.0 / (1.0 + jnp.exp(-o))


def _tc_mlp(p0, p1, W1, b1, W2, b2):
  nb = 8
  bm = BATCH // nb
  return pl.pallas_call(
      _mlp_body,
      grid=(nb,),
      in_specs=[
          pl.BlockSpec((bm, EMBED_DIM), lambda i: (i, 0)),
          pl.BlockSpec((bm, EMBED_DIM), lambda i: (i, 0)),
          pl.BlockSpec((HIDDEN_DIM, EMBED_DIM), lambda i: (0, 0)),
          pl.BlockSpec((1, HIDDEN_DIM), lambda i: (0, 0)),
          pl.BlockSpec((1, HIDDEN_DIM), lambda i: (0, 0)),
          pl.BlockSpec((1, 1), lambda i: (0, 0)),
      ],
      out_specs=pl.BlockSpec((bm, 1), lambda i: (i, 0)),
      out_shape=jax.ShapeDtypeStruct((BATCH, 1), jnp.float32),
  )(p0, p1, W1, b1, W2, b2)


@jax.jit
def kernel(x, table, W1, b1, W2, b2):
  xf = jnp.reshape(x, (BATCH * SEQ,))
  tableT = jnp.transpose(table)
  tp0 = _tc_repack(tableT, 0)
  p0 = _sc_pool_phase(xf, tp0, False)
  tp1 = _tc_repack(tableT, 2 * _NBLK)
  p1 = _sc_pool_phase(xf, tp1, True)
  out = _tc_mlp(p0, p1, W1, b1.reshape(1, HIDDEN_DIM), W2, b2.reshape(1, 1))
  return jnp.squeeze(out, axis=1)


# speedup vs baseline: 4.4952x; 4.4952x over previous
import functools

import jax
import jax.numpy as jnp
from jax import lax
from jax.experimental import pallas as pl
from jax.experimental.pallas import tpu as pltpu
from jax.experimental.pallas import tpu_sc as plsc

NUM_VOCAB = 1000000
EMBED_DIM = 64
ROW = 128
HIDDEN_DIM = 256
BATCH = 4096
SEQ = 200

_TBLK = 8192
_NBLK = 62
_SPLIT = _TBLK * _NBLK

_INFO = plsc.get_sparse_core_info()
_NC = _INFO.num_cores
_NS = _INFO.num_subcores
_NW = _NC * _NS
_BPW = BATCH // _NW
_IPW = _BPW * SEQ
_SLICES = [(0, 128), (128, 72)]


def _repack_body(x1_ref, x2_ref, o_ref):
  eye = jnp.asarray(
      lax.broadcasted_iota(jnp.int32, (EMBED_DIM, EMBED_DIM), 0)
      == lax.broadcasted_iota(jnp.int32, (EMBED_DIM, EMBED_DIM), 1),
      jnp.float32,
  )
  dims = (((0,), (0,)), ((), ()))
  o_ref[:, 0:EMBED_DIM] = lax.dot_general(
      x1_ref[...], eye, dims, preferred_element_type=jnp.float32
  )
  o_ref[:, EMBED_DIM:ROW] = lax.dot_general(
      x2_ref[...], eye, dims, preferred_element_type=jnp.float32
  )


def _tc_repack(tableT):
  return pl.pallas_call(
      _repack_body,
      grid=(_NBLK,),
      in_specs=[
          pl.BlockSpec((EMBED_DIM, _TBLK), lambda i: (0, i)),
          pl.BlockSpec(
              (EMBED_DIM, _TBLK),
              lambda i: (0, jnp.minimum(_NBLK + i, NUM_VOCAB // _TBLK)),
          ),
      ],
      out_specs=pl.BlockSpec((_TBLK, ROW), lambda i: (i, 0)),
      out_shape=jax.ShapeDtypeStruct((_SPLIT, ROW), jnp.float32),
      compiler_params=pltpu.CompilerParams(fuse_transposed_lhs_in_matmul=True),
  )(tableT, tableT)


def _sc_pool(xf, tp):
  mesh = plsc.VectorSubcoreMesh(core_axis_name="c", subcore_axis_name="s")

  @functools.partial(
      pl.kernel,
      out_type=jax.ShapeDtypeStruct((BATCH, EMBED_DIM), jnp.float32),
      mesh=mesh,
      scratch_types=[
          pltpu.VMEM((_IPW + 16,), jnp.int32),
          pltpu.VMEM((_IPW,), jnp.int32),
          pltpu.VMEM((SEQ, EMBED_DIM), jnp.float32),
          pltpu.VMEM((SEQ, EMBED_DIM), jnp.float32),
          pltpu.VMEM((_BPW, EMBED_DIM), jnp.float32),
          pltpu.SemaphoreType.DMA,
          pltpu.SemaphoreType.DMA,
      ],
      compiler_params=pltpu.CompilerParams(use_tc_tiling_on_sc=False),
  )
  def k(xf_hbm, tp_hbm, out_hbm, raw_v, idx_v, rows0, rows1, out_v,
        sem0, sem1):
    wid = lax.axis_index("s") * _NC + lax.axis_index("c")
    base = wid * _BPW
    bufs = ((rows0, sem0), (rows1, sem1))

    pltpu.sync_copy(xf_hbm.at[pl.ds(base * SEQ, _IPW)],
                    raw_v.at[pl.ds(0, _IPW)])

    @plsc.parallel_loop(0, _IPW // 16, unroll=8)
    def _(j):
      v = raw_v[pl.ds(j * 16, 16)]
      idx_v[pl.ds(j * 16, 16)] = jnp.where(
          v >= _SPLIT, 2 * (v - _SPLIT) + 1, 2 * v
      )

    def start(c, rows_v, sem):
      for o, l in _SLICES:
        pltpu.async_copy(
            tp_hbm.at[idx_v.at[pl.ds(c * SEQ + o, l)]],
            rows_v.at[pl.ds(o, l)], sem,
        )

    def finish(c, rows_v, sem):
      for o, l in _SLICES:
        pltpu.make_async_copy(
            tp_hbm.at[idx_v.at[pl.ds(c * SEQ + o, l)]],
            rows_v.at[pl.ds(o, l)], sem,
        ).wait()
      zero = jnp.zeros((16,), jnp.float32)

      @plsc.parallel_loop(0, SEQ, unroll=8, carry=(zero, zero, zero, zero))
      def accs(r, acc):
        return tuple(
            acc[g] + rows_v[r, pl.ds(g * 16, 16)] for g in range(4)
        )

      for g in range(4):
        out_v[c, pl.ds(g * 16, 16)] = accs[g] * (1.0 / SEQ)

    for b in range(2):
      start(b, *bufs[b])

    def chunk_body(g, carry):
      for b in range(2):
        c = 2 * g + b
        rows_v, sem = bufs[b]
        finish(c, rows_v, sem)

        @pl.when(c + 2 < _BPW)
        def _():
          start(c + 2, rows_v, sem)

      return carry

    lax.fori_loop(0, _BPW // 2, chunk_body, 0)
    pltpu.sync_copy(out_v, out_hbm.at[pl.ds(base, _BPW)])

  return k(xf, tp)


def _mlp_body(h0_ref, w1_ref, b1_ref, w2_ref, b2_ref, o_ref):
  h = h0_ref[...]
  h1 = lax.dot_general(
      h, w1_ref[...], (((1,), (1,)), ((), ())),
      preferred_element_type=jnp.float32,
  )
  h1 = jnp.maximum(h1 + b1_ref[...], 0.0)
  o = jnp.sum(h1 * w2_ref[...], axis=1, keepdims=True) + b2_ref[...]
  o_ref[...] = 1.0 / (1.0 + jnp.exp(-o))


def _tc_mlp(h0, W1, b1, W2, b2):
  nb = 8
  bm = BATCH // nb
  return pl.pallas_call(
      _mlp_body,
      grid=(nb,),
      in_specs=[
          pl.BlockSpec((bm, EMBED_DIM), lambda i: (i, 0)),
          pl.BlockSpec((HIDDEN_DIM, EMBED_DIM), lambda i: (0, 0)),
          pl.BlockSpec((1, HIDDEN_DIM), lambda i: (0, 0)),
          pl.BlockSpec((1, HIDDEN_DIM), lambda i: (0, 0)),
          pl.BlockSpec((1, 1), lambda i: (0, 0)),
      ],
      out_specs=pl.BlockSpec((bm, 1), lambda i: (i, 0)),
      out_shape=jax.ShapeDtypeStruct((BATCH, 1), jnp.float32),
  )(h0, W1, b1, W2, b2)


@jax.jit
def kernel(x, table, W1, b1, W2, b2):
  xf = jnp.reshape(x, (BATCH * SEQ,))
  tp = _tc_repack(jnp.transpose(table))
  h0 = _sc_pool(xf, jnp.reshape(tp, (2 * _SPLIT, EMBED_DIM)))
  out = _tc_mlp(h0, W1, b1.reshape(1, HIDDEN_DIM), W2, b2.reshape(1, 1))
  return jnp.squeeze(out, axis=1)


# speedup vs baseline: 4.6685x; 1.0385x over previous
import functools

import jax
import jax.numpy as jnp
from jax import lax
from jax.experimental import pallas as pl
from jax.experimental.pallas import tpu as pltpu
from jax.experimental.pallas import tpu_sc as plsc

NUM_VOCAB = 1000000
EMBED_DIM = 64
ROW = 128
HIDDEN_DIM = 256
BATCH = 4096
SEQ = 200

_TBLK = 16384
_NBLK = 31
_SPLIT = _TBLK * _NBLK

_INFO = plsc.get_sparse_core_info()
_NC = _INFO.num_cores
_NS = _INFO.num_subcores
_NW = _NC * _NS
_BPW = BATCH // _NW
_IPW = _BPW * SEQ
_SLICES = [(0, 128), (128, 72)]


def _repack_body(x1_ref, x2_ref, o_ref):
  eye = jnp.asarray(
      lax.broadcasted_iota(jnp.int32, (EMBED_DIM, EMBED_DIM), 0)
      == lax.broadcasted_iota(jnp.int32, (EMBED_DIM, EMBED_DIM), 1),
      jnp.float32,
  )
  del eye
  o_ref[:, 0:EMBED_DIM] = jnp.transpose(x1_ref[...])
  o_ref[:, EMBED_DIM:ROW] = jnp.transpose(x2_ref[...])


def _tc_repack(tableT):
  return pl.pallas_call(
      _repack_body,
      grid=(_NBLK,),
      in_specs=[
          pl.BlockSpec((EMBED_DIM, _TBLK), lambda i: (0, i)),
          pl.BlockSpec(
              (EMBED_DIM, _TBLK),
              lambda i: (0, jnp.minimum(_NBLK + i, NUM_VOCAB // _TBLK)),
          ),
      ],
      out_specs=pl.BlockSpec((_TBLK, ROW), lambda i: (i, 0)),
      out_shape=jax.ShapeDtypeStruct((_SPLIT, ROW), jnp.float32),
      compiler_params=pltpu.CompilerParams(fuse_transposed_lhs_in_matmul=True),
  )(tableT, tableT)


def _sc_pool(xf, tp):
  mesh = plsc.VectorSubcoreMesh(core_axis_name="c", subcore_axis_name="s")

  @functools.partial(
      pl.kernel,
      out_type=jax.ShapeDtypeStruct((BATCH, EMBED_DIM), jnp.float32),
      mesh=mesh,
      scratch_types=[
          pltpu.VMEM((_IPW + 16,), jnp.int32),
          pltpu.VMEM((_IPW,), jnp.int32),
          pltpu.VMEM((SEQ, EMBED_DIM), jnp.float32),
          pltpu.VMEM((SEQ, EMBED_DIM), jnp.float32),
          pltpu.VMEM((_BPW, EMBED_DIM), jnp.float32),
          pltpu.SemaphoreType.DMA,
          pltpu.SemaphoreType.DMA,
      ],
      compiler_params=pltpu.CompilerParams(use_tc_tiling_on_sc=False),
  )
  def k(xf_hbm, tp_hbm, out_hbm, raw_v, idx_v, rows0, rows1, out_v,
        sem0, sem1):
    wid = lax.axis_index("s") * _NC + lax.axis_index("c")
    base = wid * _BPW
    bufs = ((rows0, sem0), (rows1, sem1))

    pltpu.sync_copy(xf_hbm.at[pl.ds(base * SEQ, _IPW)],
                    raw_v.at[pl.ds(0, _IPW)])

    @plsc.parallel_loop(0, _IPW // 16, unroll=8)
    def _(j):
      v = raw_v[pl.ds(j * 16, 16)]
      idx_v[pl.ds(j * 16, 16)] = jnp.where(
          v >= _SPLIT, 2 * (v - _SPLIT) + 1, 2 * v
      )

    def start(c, rows_v, sem):
      for o, l in _SLICES:
        pltpu.async_copy(
            tp_hbm.at[idx_v.at[pl.ds(c * SEQ + o, l)]],
            rows_v.at[pl.ds(o, l)], sem,
        )

    def finish(c, rows_v, sem):
      for o, l in _SLICES:
        pltpu.make_async_copy(
            tp_hbm.at[idx_v.at[pl.ds(c * SEQ + o, l)]],
            rows_v.at[pl.ds(o, l)], sem,
        ).wait()
      zero = jnp.zeros((16,), jnp.float32)

      @plsc.parallel_loop(0, SEQ, unroll=8, carry=(zero, zero, zero, zero))
      def accs(r, acc):
        return tuple(
            acc[g] + rows_v[r, pl.ds(g * 16, 16)] for g in range(4)
        )

      for g in range(4):
        out_v[c, pl.ds(g * 16, 16)] = accs[g] * (1.0 / SEQ)

    for b in range(2):
      start(b, *bufs[b])

    def chunk_body(g, carry):
      for b in range(2):
        c = 2 * g + b
        rows_v, sem = bufs[b]
        finish(c, rows_v, sem)

        @pl.when(c + 2 < _BPW)
        def _():
          start(c + 2, rows_v, sem)

      return carry

    lax.fori_loop(0, _BPW // 2, chunk_body, 0)
    pltpu.sync_copy(out_v, out_hbm.at[pl.ds(base, _BPW)])

  return k(xf, tp)


def _mlp_body(h0_ref, w1_ref, b1_ref, w2_ref, b2_ref, o_ref):
  h = h0_ref[...]
  h1 = lax.dot_general(
      h, w1_ref[...], (((1,), (1,)), ((), ())),
      preferred_element_type=jnp.float32,
  )
  h1 = jnp.maximum(h1 + b1_ref[...], 0.0)
  o = jnp.sum(h1 * w2_ref[...], axis=1, keepdims=True) + b2_ref[...]
  o_ref[...] = 1.0 / (1.0 + jnp.exp(-o))


def _tc_mlp(h0, W1, b1, W2, b2):
  nb = 8
  bm = BATCH // nb
  return pl.pallas_call(
      _mlp_body,
      grid=(nb,),
      in_specs=[
          pl.BlockSpec((bm, EMBED_DIM), lambda i: (i, 0)),
          pl.BlockSpec((HIDDEN_DIM, EMBED_DIM), lambda i: (0, 0)),
          pl.BlockSpec((1, HIDDEN_DIM), lambda i: (0, 0)),
          pl.BlockSpec((1, HIDDEN_DIM), lambda i: (0, 0)),
          pl.BlockSpec((1, 1), lambda i: (0, 0)),
      ],
      out_specs=pl.BlockSpec((bm, 1), lambda i: (i, 0)),
      out_shape=jax.ShapeDtypeStruct((BATCH, 1), jnp.float32),
  )(h0, W1, b1, W2, b2)


@jax.jit
def kernel(x, table, W1, b1, W2, b2):
  xf = jnp.reshape(x, (BATCH * SEQ,))
  tp = _tc_repack(jnp.transpose(table))
  h0 = _sc_pool(xf, jnp.reshape(tp, (2 * _SPLIT, EMBED_DIM)))
  out = _tc_mlp(h0, W1, b1.reshape(1, HIDDEN_DIM), W2, b2.reshape(1, 1))
  return jnp.squeeze(out, axis=1)


# speedup vs baseline: 4.9196x; 1.0538x over previous
import functools

import jax
import jax.numpy as jnp
from jax import lax
from jax.experimental import pallas as pl
from jax.experimental.pallas import tpu as pltpu
from jax.experimental.pallas import tpu_sc as plsc

NUM_VOCAB = 1000000
EMBED_DIM = 64
ROW = 128
HIDDEN_DIM = 256
BATCH = 4096
SEQ = 200

_TBLK = 16384
_NBLK = 31
_SPLIT = _TBLK * _NBLK

_INFO = plsc.get_sparse_core_info()
_NC = _INFO.num_cores
_NS = _INFO.num_subcores
_NW = _NC * _NS
_BPW = BATCH // _NW
_IPW = _BPW * SEQ
_CB = 2
_CSEQ = _CB * SEQ
_SLICES = [(0, 128), (128, 128), (256, 128), (384, 16)]


def _repack_body(x1_ref, x2_ref, o_ref):
  eye = jnp.asarray(
      lax.broadcasted_iota(jnp.int32, (EMBED_DIM, EMBED_DIM), 0)
      == lax.broadcasted_iota(jnp.int32, (EMBED_DIM, EMBED_DIM), 1),
      jnp.float32,
  )
  del eye
  o_ref[:, 0:EMBED_DIM] = jnp.transpose(x1_ref[...])
  o_ref[:, EMBED_DIM:ROW] = jnp.transpose(x2_ref[...])


def _tc_repack(tableT):
  return pl.pallas_call(
      _repack_body,
      grid=(_NBLK,),
      in_specs=[
          pl.BlockSpec((EMBED_DIM, _TBLK), lambda i: (0, i)),
          pl.BlockSpec(
              (EMBED_DIM, _TBLK),
              lambda i: (0, jnp.minimum(_NBLK + i, NUM_VOCAB // _TBLK)),
          ),
      ],
      out_specs=pl.BlockSpec((_TBLK, ROW), lambda i: (i, 0)),
      out_shape=jax.ShapeDtypeStruct((_SPLIT, ROW), jnp.float32),
      compiler_params=pltpu.CompilerParams(fuse_transposed_lhs_in_matmul=True),
  )(tableT, tableT)


def _sc_pool(xf, tp):
  mesh = plsc.VectorSubcoreMesh(core_axis_name="c", subcore_axis_name="s")

  @functools.partial(
      pl.kernel,
      out_type=jax.ShapeDtypeStruct((BATCH, EMBED_DIM), jnp.float32),
      mesh=mesh,
      scratch_types=[
          pltpu.VMEM((_IPW + 16,), jnp.int32),
          pltpu.VMEM((_IPW,), jnp.int32),
          pltpu.VMEM((_CSEQ, EMBED_DIM), jnp.float32),
          pltpu.VMEM((_CSEQ, EMBED_DIM), jnp.float32),
          pltpu.VMEM((_BPW, EMBED_DIM), jnp.float32),
          pltpu.SemaphoreType.DMA,
          pltpu.SemaphoreType.DMA,
      ],
      compiler_params=pltpu.CompilerParams(use_tc_tiling_on_sc=False),
  )
  def k(xf_hbm, tp_hbm, out_hbm, raw_v, idx_v, rows0, rows1, out_v,
        sem0, sem1):
    wid = lax.axis_index("s") * _NC + lax.axis_index("c")
    base = wid * _BPW
    bufs = ((rows0, sem0), (rows1, sem1))

    pltpu.sync_copy(xf_hbm.at[pl.ds(base * SEQ, _IPW)],
                    raw_v.at[pl.ds(0, _IPW)])

    @plsc.parallel_loop(0, _IPW // 16, unroll=8)
    def _(j):
      v = raw_v[pl.ds(j * 16, 16)]
      idx_v[pl.ds(j * 16, 16)] = jnp.where(
          v >= _SPLIT, 2 * (v - _SPLIT) + 1, 2 * v
      )

    def start(c, rows_v, sem):
      for o, l in _SLICES:
        pltpu.async_copy(
            tp_hbm.at[idx_v.at[pl.ds(c * _CSEQ + o, l)]],
            rows_v.at[pl.ds(o, l)], sem,
        )

    def finish(c, rows_v, sem):
      for o, l in _SLICES:
        pltpu.make_async_copy(
            tp_hbm.at[idx_v.at[pl.ds(c * _CSEQ + o, l)]],
            rows_v.at[pl.ds(o, l)], sem,
        ).wait()
      zero = jnp.zeros((16,), jnp.float32)
      for e in range(_CB):

        @plsc.parallel_loop(0, SEQ, unroll=8, carry=(zero, zero, zero, zero))
        def accs(r, acc):
          return tuple(
              acc[g] + rows_v[e * SEQ + r, pl.ds(g * 16, 16)]
              for g in range(4)
          )

        for g in range(4):
          out_v[c * _CB + e, pl.ds(g * 16, 16)] = accs[g] * (1.0 / SEQ)

    nchunk = _BPW // _CB
    for b in range(2):
      start(b, *bufs[b])

    def chunk_body(g, carry):
      for b in range(2):
        c = 2 * g + b
        rows_v, sem = bufs[b]
        finish(c, rows_v, sem)

        @pl.when(c + 2 < nchunk)
        def _():
          start(c + 2, rows_v, sem)

      return carry

    lax.fori_loop(0, nchunk // 2, chunk_body, 0)
    pltpu.sync_copy(out_v, out_hbm.at[pl.ds(base, _BPW)])

  return k(xf, tp)


def _mlp_body(h0_ref, w1_ref, b1_ref, w2_ref, b2_ref, o_ref):
  h = h0_ref[...]
  h1 = lax.dot_general(
      h, w1_ref[...], (((1,), (1,)), ((), ())),
      preferred_element_type=jnp.float32,
  )
  h1 = jnp.maximum(h1 + b1_ref[...], 0.0)
  o = jnp.sum(h1 * w2_ref[...], axis=1, keepdims=True) + b2_ref[...]
  o_ref[...] = 1.0 / (1.0 + jnp.exp(-o))


def _tc_mlp(h0, W1, b1, W2, b2):
  nb = 8
  bm = BATCH // nb
  return pl.pallas_call(
      _mlp_body,
      grid=(nb,),
      in_specs=[
          pl.BlockSpec((bm, EMBED_DIM), lambda i: (i, 0)),
          pl.BlockSpec((HIDDEN_DIM, EMBED_DIM), lambda i: (0, 0)),
          pl.BlockSpec((1, HIDDEN_DIM), lambda i: (0, 0)),
          pl.BlockSpec((1, HIDDEN_DIM), lambda i: (0, 0)),
          pl.BlockSpec((1, 1), lambda i: (0, 0)),
      ],
      out_specs=pl.BlockSpec((bm, 1), lambda i: (i, 0)),
      out_shape=jax.ShapeDtypeStruct((BATCH, 1), jnp.float32),
  )(h0, W1, b1, W2, b2)


@jax.jit
def kernel(x, table, W1, b1, W2, b2):
  xf = jnp.reshape(x, (BATCH * SEQ,))
  tp = _tc_repack(jnp.transpose(table))
  h0 = _sc_pool(xf, jnp.reshape(tp, (2 * _SPLIT, EMBED_DIM)))
  out = _tc_mlp(h0, W1, b1.reshape(1, HIDDEN_DIM), W2, b2.reshape(1, 1))
  return jnp.squeeze(out, axis=1)


# speedup vs baseline: 5.0361x; 1.0237x over previous
import functools

import jax
import jax.numpy as jnp
from jax import lax
from jax.experimental import pallas as pl
from jax.experimental.pallas import tpu as pltpu
from jax.experimental.pallas import tpu_sc as plsc

NUM_VOCAB = 1000000
EMBED_DIM = 64
ROW = 128
HIDDEN_DIM = 256
BATCH = 4096
SEQ = 200

_TBLK = 16384
_NBLK = 31
_SPLIT = _TBLK * _NBLK

_INFO = plsc.get_sparse_core_info()
_NC = _INFO.num_cores
_NS = _INFO.num_subcores
_NW = _NC * _NS
_BPW = BATCH // _NW
_IPW = _BPW * SEQ
_CB = 2
_CSEQ = _CB * SEQ
_SLICES = [(0, 128), (128, 128), (256, 128), (384, 16)]


def _repack_body(x1_ref, x2_ref, o_ref):
  eye = jnp.asarray(
      lax.broadcasted_iota(jnp.int32, (EMBED_DIM, EMBED_DIM), 0)
      == lax.broadcasted_iota(jnp.int32, (EMBED_DIM, EMBED_DIM), 1),
      jnp.float32,
  )
  del eye
  o_ref[:, 0:EMBED_DIM] = jnp.transpose(x1_ref[...])
  o_ref[:, EMBED_DIM:ROW] = jnp.transpose(x2_ref[...])


def _tc_repack(tableT):
  return pl.pallas_call(
      _repack_body,
      grid=(_NBLK,),
      in_specs=[
          pl.BlockSpec((EMBED_DIM, _TBLK), lambda i: (0, i)),
          pl.BlockSpec(
              (EMBED_DIM, _TBLK),
              lambda i: (0, jnp.minimum(_NBLK + i, NUM_VOCAB // _TBLK)),
          ),
      ],
      out_specs=pl.BlockSpec((_TBLK, ROW), lambda i: (i, 0)),
      out_shape=jax.ShapeDtypeStruct((_SPLIT, ROW), jnp.float32),
      compiler_params=pltpu.CompilerParams(fuse_transposed_lhs_in_matmul=True),
  )(tableT, tableT)


def _sc_pool(xf, tp):
  mesh = plsc.VectorSubcoreMesh(core_axis_name="c", subcore_axis_name="s")

  @functools.partial(
      pl.kernel,
      out_type=jax.ShapeDtypeStruct((BATCH, EMBED_DIM), jnp.float32),
      mesh=mesh,
      scratch_types=[
          pltpu.VMEM((_IPW + 16,), jnp.int32),
          pltpu.VMEM((_CSEQ, EMBED_DIM), jnp.float32),
          pltpu.VMEM((_CSEQ, EMBED_DIM), jnp.float32),
          pltpu.VMEM((_CSEQ, EMBED_DIM), jnp.float32),
          pltpu.VMEM((_BPW, EMBED_DIM), jnp.float32),
          pltpu.SemaphoreType.DMA,
          pltpu.SemaphoreType.DMA,
          pltpu.SemaphoreType.DMA,
      ],
      compiler_params=pltpu.CompilerParams(use_tc_tiling_on_sc=False),
  )
  def k(xf_hbm, tp_hbm, out_hbm, idx_v, rows0, rows1, rows2, out_v,
        sem0, sem1, sem2):
    wid = lax.axis_index("s") * _NC + lax.axis_index("c")
    base = wid * _BPW
    bufs = ((rows0, sem0), (rows1, sem1), (rows2, sem2))

    pltpu.sync_copy(xf_hbm.at[pl.ds(base * SEQ, _IPW)],
                    idx_v.at[pl.ds(0, _IPW)])

    @plsc.parallel_loop(0, _IPW // 16, unroll=8)
    def _(j):
      v = idx_v[pl.ds(j * 16, 16)]
      idx_v[pl.ds(j * 16, 16)] = jnp.where(
          v >= _SPLIT, 2 * (v - _SPLIT) + 1, 2 * v
      )

    def start(c, rows_v, sem):
      for o, l in _SLICES:
        pltpu.async_copy(
            tp_hbm.at[idx_v.at[pl.ds(c * _CSEQ + o, l)]],
            rows_v.at[pl.ds(o, l)], sem,
        )

    def finish(c, rows_v, sem):
      for o, l in _SLICES:
        pltpu.make_async_copy(
            tp_hbm.at[idx_v.at[pl.ds(c * _CSEQ + o, l)]],
            rows_v.at[pl.ds(o, l)], sem,
        ).wait()
      zero = jnp.zeros((16,), jnp.float32)
      for e in range(_CB):

        @plsc.parallel_loop(0, SEQ, unroll=8, carry=(zero, zero, zero, zero))
        def accs(r, acc):
          return tuple(
              acc[g] + rows_v[e * SEQ + r, pl.ds(g * 16, 16)]
              for g in range(4)
          )

        for g in range(4):
          out_v[c * _CB + e, pl.ds(g * 16, 16)] = accs[g] * (1.0 / SEQ)

    nchunk = _BPW // _CB
    for b in range(3):
      start(b, *bufs[b])

    def chunk_body(g, carry):
      for b in range(3):
        c = 3 * g + b
        rows_v, sem = bufs[b]

        @pl.when(c < nchunk)
        def _():
          finish(c, rows_v, sem)

        @pl.when(c + 3 < nchunk)
        def _():
          start(c + 3, rows_v, sem)

      return carry

    lax.fori_loop(0, (nchunk + 2) // 3, chunk_body, 0)
    pltpu.sync_copy(out_v, out_hbm.at[pl.ds(base, _BPW)])

  return k(xf, tp)


def _mlp_body(h0_ref, w1_ref, b1_ref, w2_ref, b2_ref, o_ref):
  h = h0_ref[...]
  h1 = lax.dot_general(
      h, w1_ref[...], (((1,), (1,)), ((), ())),
      preferred_element_type=jnp.float32,
  )
  h1 = jnp.maximum(h1 + b1_ref[...], 0.0)
  o = jnp.sum(h1 * w2_ref[...], axis=1, keepdims=True) + b2_ref[...]
  o_ref[...] = 1.0 / (1.0 + jnp.exp(-o))


def _tc_mlp(h0, W1, b1, W2, b2):
  nb = 8
  bm = BATCH // nb
  return pl.pallas_call(
      _mlp_body,
      grid=(nb,),
      in_specs=[
          pl.BlockSpec((bm, EMBED_DIM), lambda i: (i, 0)),
          pl.BlockSpec((HIDDEN_DIM, EMBED_DIM), lambda i: (0, 0)),
          pl.BlockSpec((1, HIDDEN_DIM), lambda i: (0, 0)),
          pl.BlockSpec((1, HIDDEN_DIM), lambda i: (0, 0)),
          pl.BlockSpec((1, 1), lambda i: (0, 0)),
      ],
      out_specs=pl.BlockSpec((bm, 1), lambda i: (i, 0)),
      out_shape=jax.ShapeDtypeStruct((BATCH, 1), jnp.float32),
  )(h0, W1, b1, W2, b2)


@jax.jit
def kernel(x, table, W1, b1, W2, b2):
  xf = jnp.reshape(x, (BATCH * SEQ,))
  tp = _tc_repack(jnp.transpose(table))
  h0 = _sc_pool(xf, jnp.reshape(tp, (2 * _SPLIT, EMBED_DIM)))
  out = _tc_mlp(h0, W1, b1.reshape(1, HIDDEN_DIM), W2, b2.reshape(1, 1))
  return jnp.squeeze(out, axis=1)
